# baseline (device time: 43926 ns/iter reference)
import jax
import jax.numpy as jnp
from jax import lax
from jax.experimental import pallas as pl
from jax.experimental.pallas import tpu as pltpu

N_DEV = 32
K_CHUNK = 512
N_CHUNKS = 4096 // K_CHUNK
TILES_PER_CHUNK = K_CHUNK // 128


def kernel(x, w_mat):
    m_total, k_per = x.shape
    k_total, n_out = w_mat.shape
    m_per = m_total // N_DEV

    def body(x_ref, w_ref, out_ref,
             xbf_ref, xstage_ref, wbuf_ref, amax_ref,
             wdma_sems, send_sems, recv_sems, amax_send_sems,
             amax_recv_sems):
        me = lax.axis_index("i")

        def w_dma(j):
            return pltpu.make_async_copy(
                w_ref.at[pl.ds(j * K_CHUNK, K_CHUNK), :],
                wbuf_ref.at[j % 2],
                wdma_sems.at[j % 2],
            )
        w_dma(0).start()
        w_dma(1).start()

        xbf_ref[...] = x_ref[...].astype(jnp.bfloat16)

        barrier_sem = pltpu.get_barrier_semaphore()
        for d in range(1, N_DEV):
            pl.semaphore_signal(
                barrier_sem, inc=1,
                device_id=((me + d) % N_DEV,),
                device_id_type=pl.DeviceIdType.MESH,
            )
        pl.semaphore_wait(barrier_sem, N_DEV - 1)

        for d in range(N_DEV):
            p = (me + d) % N_DEV
            rdma = pltpu.make_async_remote_copy(
                src_ref=xbf_ref.at[pl.ds(p * m_per, m_per), :],
                dst_ref=xstage_ref.at[me],
                send_sem=send_sems.at[p],
                recv_sem=recv_sems.at[me],
                device_id=(p,),
                device_id_type=pl.DeviceIdType.MESH,
            )
            rdma.start()

        def recv_wait(q):
            recv = pltpu.make_async_remote_copy(
                src_ref=xbf_ref.at[pl.ds(q * m_per, m_per), :],
                dst_ref=xstage_ref.at[q],
                send_sem=send_sems.at[q],
                recv_sem=recv_sems.at[q],
                device_id=(q,),
                device_id_type=pl.DeviceIdType.MESH,
            )
            recv.wait_recv()

        y = jnp.zeros((m_per, n_out), jnp.float32)
        for j in range(N_CHUNKS):
            w_dma(j).wait()
            for t in range(TILES_PER_CHUNK):
                recv_wait(j * TILES_PER_CHUNK + t)

            xk = jnp.concatenate(
                [xstage_ref[j * TILES_PER_CHUNK + t]
                 for t in range(TILES_PER_CHUNK)],
                axis=1,
            )
            wk = wbuf_ref[j % 2].astype(jnp.bfloat16)
            y = y + lax.dot_general(
                xk, wk,
                dimension_numbers=(((1,), (0,)), ((), ())),
                preferred_element_type=jnp.float32,
            )
            if j + 2 < N_CHUNKS:
                w_dma(j + 2).start()

        local_amax = jnp.max(jnp.abs(y))
        amax_ref[me] = jnp.full((8, 128), local_amax, jnp.float32)
        for d in range(1, N_DEV):
            p = (me + d) % N_DEV
            rdma = pltpu.make_async_remote_copy(
                src_ref=amax_ref.at[me],
                dst_ref=amax_ref.at[me],
                send_sem=amax_send_sems.at[p],
                recv_sem=amax_recv_sems.at[me],
                device_id=(p,),
                device_id_type=pl.DeviceIdType.MESH,
            )
            rdma.start()
        for d in range(1, N_DEV):
            q = (me - d) % N_DEV
            recv = pltpu.make_async_remote_copy(
                src_ref=amax_ref.at[q],
                dst_ref=amax_ref.at[q],
                send_sem=amax_send_sems.at[q],
                recv_sem=amax_recv_sems.at[q],
                device_id=(q,),
                device_id_type=pl.DeviceIdType.MESH,
            )
            recv.wait_recv()
        global_amax = jnp.max(amax_ref[...])

        scale = global_amax / 127.0
        qv = jnp.clip(jnp.round(y / scale), -127.0, 127.0)
        out_ref[...] = qv * scale

        for d in range(N_DEV):
            p = (me + d) % N_DEV
            s = pltpu.make_async_remote_copy(
                src_ref=xbf_ref.at[pl.ds(p * m_per, m_per), :],
                dst_ref=xstage_ref.at[me],
                send_sem=send_sems.at[p],
                recv_sem=recv_sems.at[me],
                device_id=(p,),
                device_id_type=pl.DeviceIdType.MESH,
            )
            s.wait_send()
        for d in range(1, N_DEV):
            p = (me + d) % N_DEV
            s2 = pltpu.make_async_remote_copy(
                src_ref=amax_ref.at[me],
                dst_ref=amax_ref.at[me],
                send_sem=amax_send_sems.at[p],
                recv_sem=amax_recv_sems.at[me],
                device_id=(p,),
                device_id_type=pl.DeviceIdType.MESH,
            )
            s2.wait_send()

    return pl.pallas_call(
        body,
        out_shape=jax.ShapeDtypeStruct((m_per, n_out), jnp.float32),
        in_specs=[
            pl.BlockSpec(memory_space=pltpu.VMEM),
            pl.BlockSpec(memory_space=pltpu.MemorySpace.HBM),
        ],
        out_specs=pl.BlockSpec(memory_space=pltpu.VMEM),
        scratch_shapes=[
            pltpu.VMEM((m_total, k_per), jnp.bfloat16),
            pltpu.VMEM((N_DEV, m_per, k_per), jnp.bfloat16),
            pltpu.VMEM((2, K_CHUNK, n_out), jnp.float32),
            pltpu.VMEM((N_DEV, 8, 128), jnp.float32),
            pltpu.SemaphoreType.DMA((2,)),
            pltpu.SemaphoreType.DMA((N_DEV,)),
            pltpu.SemaphoreType.DMA((N_DEV,)),
            pltpu.SemaphoreType.DMA((N_DEV,)),
            pltpu.SemaphoreType.DMA((N_DEV,)),
        ],
        compiler_params=pltpu.CompilerParams(
            collective_id=0,
            vmem_limit_bytes=60 * 1024 * 1024,
        ),
    )(x, w_mat)
